# Initial kernel scaffold; baseline (speedup 1.0000x reference)
import jax
import jax.numpy as jnp
from jax.experimental import pallas as pl


def kernel(features, Wxf, bxf, Wlf, blf, Wrf, brf, Wxb, bxb, Whb, bhb):
    raise NotImplementedError("write your pallas kernel here")


# speedup vs baseline: 31.9510x; 31.9510x over previous
import jax
import jax.numpy as jnp
from jax.experimental import pallas as pl
from jax.experimental.pallas import tpu as pltpu

_N = 10000
_D = 128
_H = 64
_L = 2
_PAD = 16384


def _level_bounds(n):
    levs = []
    start, size = 0, 1
    while start < n:
        levs.append((start, min(start + size, n)))
        start += size
        size *= 2
    return levs


_LEVELS = _level_bounds(_N)


def _btree_kernel(feat_ref, wxf_ref, wlf_ref, wrf_ref, bxf_ref, bff_ref,
                  blf_ref, brf_ref, wxb_ref, whb_ref, bbb_ref, out_ref,
                  x1_ref, h_ref, c_ref):
    f32 = jnp.float32
    h_ref[_N:_PAD, :] = jnp.zeros((_PAD - _N, _H), f32)
    c_ref[_N:_PAD, :] = jnp.zeros((_PAD - _N, _H), f32)

    leaf_start = _LEVELS[-1][0]

    def fwd(l, x_ref, dst_ref):
        wx = wxf_ref[l]
        wl = wlf_ref[l]
        wr = wrf_ref[l]
        for (s, e) in reversed(_LEVELS):
            m = e - s
            g = jnp.dot(x_ref[s:e, :], wx, preferred_element_type=f32)
            if s == leaf_start:
                g = g + bxf_ref[l]
                cg = jax.nn.sigmoid(g[:, 0:_H]) * jnp.tanh(g[:, 4 * _H:])
            else:
                cs = 2 * s + 1
                lh = h_ref[pl.Slice(cs, m, 2), :]
                rh = h_ref[pl.Slice(cs + 1, m, 2), :]
                lc = c_ref[pl.Slice(cs, m, 2), :]
                rc = c_ref[pl.Slice(cs + 1, m, 2), :]
                g = g + jnp.dot(lh, wl, preferred_element_type=f32)
                g = g + jnp.dot(rh, wr, preferred_element_type=f32)
                n_l = min(max(5000 - s, 0), m)
                n_r = min(max(4999 - s, 0), m)
                if n_l == m and n_r == m:
                    g = g + bff_ref[l]
                else:
                    g = g + bxf_ref[l]
                    row = jax.lax.broadcasted_iota(jnp.int32, (m, 1), 0)
                    if n_l > 0:
                        g = g + jnp.where(row < n_l, 1.0, 0.0) * blf_ref[l]
                    if n_r > 0:
                        g = g + jnp.where(row < n_r, 1.0, 0.0) * brf_ref[l]
                cg = (jax.nn.sigmoid(g[:, 0:_H]) * jnp.tanh(g[:, 4 * _H:])
                      + jax.nn.sigmoid(g[:, 2 * _H:3 * _H]) * lc
                      + jax.nn.sigmoid(g[:, 3 * _H:4 * _H]) * rc)
            hg = jax.nn.sigmoid(g[:, _H:2 * _H]) * jnp.tanh(cg)
            c_ref[s:e, :] = cg
            h_ref[s:e, :] = hg
            dst_ref[s:e, 0:_H] = hg

    def bwd(l, x_ref, dst_ref):
        wx = wxb_ref[l]
        wh = whb_ref[l]
        for (s, e) in _LEVELS:
            m = e - s
            if s == 0:
                g = jnp.dot(x_ref[0:1, :], wx, preferred_element_type=f32)
                g = g + bbb_ref[l]
                cg = jax.nn.sigmoid(g[:, 0:_H]) * jnp.tanh(g[:, 3 * _H:])
                hg = jax.nn.sigmoid(g[:, _H:2 * _H]) * jnp.tanh(cg)
                c_ref[0:1, :] = cg
                h_ref[0:1, :] = hg
                continue
            ps = (s - 1) // 2
            pe = (e - 2) // 2 + 1
            mp = pe - ps
            pg = jnp.dot(h_ref[ps:pe, :], wh, preferred_element_type=f32)
            pc = c_ref[ps:pe, :]
            for par, n_p in ((0, (m + 1) // 2), (1, m // 2)):
                xs = x_ref[pl.Slice(s + par, n_p, 2), :]
                g = jnp.dot(xs, wx, preferred_element_type=f32)
                g = g + pg[0:n_p, :] + bbb_ref[l]
                cg = (jax.nn.sigmoid(g[:, 0:_H]) * jnp.tanh(g[:, 3 * _H:])
                      + jax.nn.sigmoid(g[:, 2 * _H:3 * _H]) * pc[0:n_p, :])
                hg = jax.nn.sigmoid(g[:, _H:2 * _H]) * jnp.tanh(cg)
                c_ref[pl.Slice(s + par, n_p, 2), :] = cg
                h_ref[pl.Slice(s + par, n_p, 2), :] = hg
        dst_ref[0:_N, _H:2 * _H] = h_ref[0:_N, :]

    fwd(0, feat_ref, x1_ref)
    bwd(0, feat_ref, x1_ref)
    fwd(1, x1_ref, out_ref)
    bwd(1, x1_ref, out_ref)


def kernel(features, Wxf, bxf, Wlf, blf, Wrf, brf, Wxb, bxb, Whb, bhb):
    f32 = jnp.float32
    wxfT = jnp.transpose(Wxf, (0, 2, 1))
    wlfT = jnp.transpose(Wlf, (0, 2, 1))
    wrfT = jnp.transpose(Wrf, (0, 2, 1))
    wxbT = jnp.transpose(Wxb, (0, 2, 1))
    whbT = jnp.transpose(Whb, (0, 2, 1))
    bxf1 = bxf[:, None, :]
    bff = (bxf + blf + brf)[:, None, :]
    blf1 = blf[:, None, :]
    brf1 = brf[:, None, :]
    bbb = (bxb + bhb)[:, None, :]
    return pl.pallas_call(
        _btree_kernel,
        out_shape=jax.ShapeDtypeStruct((_N, _D), f32),
        scratch_shapes=[
            pltpu.VMEM((_N, _D), f32),
            pltpu.VMEM((_PAD, _H), f32),
            pltpu.VMEM((_PAD, _H), f32),
        ],
    )(features.astype(f32), wxfT, wlfT, wrfT, bxf1, bff, blf1, brf1,
      wxbT, whbT, bbb)
